# Initial kernel scaffold; baseline (speedup 1.0000x reference)
import jax
import jax.numpy as jnp
from jax.experimental import pallas as pl


def kernel(log_probs, attn, prev_attn_peak, sequence_scores):
    raise NotImplementedError("write your pallas kernel here")


# speedup vs baseline: 1.5915x; 1.5915x over previous
import jax
import jax.numpy as jnp
from jax.experimental import pallas as pl

BATCH = 16
BEAM = 8
VOCAB = 32768
ENC_LEN = 2048
EOS_INDEX = 32767
MAX_ATTN_SHIFT = 60.0
EOS_THRESHOLD = 1.5
MINUS_INF = -1e20
NEG_INF = float("-inf")
BIG_I32 = 2**30


def _step_kernel(prev_ref, seq_ref, lp_ref, attn_ref, ts_ref, tok_ref, pred_ref):
    b = pl.program_id(0)
    attn = attn_ref[...]
    am = jnp.max(attn, axis=1, keepdims=True)
    col_a = jax.lax.broadcasted_iota(jnp.int32, attn.shape, 1)
    peak = jnp.min(jnp.where(attn == am, col_a, BIG_I32), axis=1, keepdims=True)
    prev = prev_ref[...][:, 0:1]
    cond = peak.astype(jnp.float32) < prev + MAX_ATTN_SHIFT

    lp = lp_ref[...]
    lpm = jnp.where(cond, lp, jnp.float32(MINUS_INF))
    col = jax.lax.broadcasted_iota(jnp.int32, lpm.shape, 1)
    is_eos = col == EOS_INDEX
    m_nz = jnp.max(jnp.where(is_eos, jnp.float32(NEG_INF), lpm), axis=1, keepdims=True)
    eos_v = jnp.max(jnp.where(is_eos, lpm, jnp.float32(NEG_INF)), axis=1, keepdims=True)
    eos_m = jnp.where(eos_v < EOS_THRESHOLD * m_nz, jnp.float32(MINUS_INF), eos_v)

    seq = seq_ref[...][:, 0:1]
    scores = seq + jnp.where(is_eos, eos_m, lpm)
    row = jax.lax.broadcasted_iota(jnp.int32, scores.shape, 0)
    cand = row * VOCAB + col

    ts = jnp.zeros((1, BEAM), jnp.float32)
    ci = jnp.zeros((1, BEAM), jnp.int32)
    lane8 = jax.lax.broadcasted_iota(jnp.int32, (1, BEAM), 1)
    for k in range(BEAM):
        m = jnp.max(scores)
        i = jnp.min(jnp.where(scores == m, cand, BIG_I32))
        ts = jnp.where(lane8 == k, m, ts)
        ci = jnp.where(lane8 == k, i, ci)
        scores = jnp.where(cand == i, jnp.float32(NEG_INF), scores)

    ts_ref[...] = ts.reshape(1, 1, BEAM)
    tok_ref[...] = jnp.bitwise_and(ci, VOCAB - 1).reshape(1, 1, BEAM)
    pred_ref[...] = (jnp.right_shift(ci, 15) + b * BEAM).reshape(1, 1, BEAM)


@jax.jit
def kernel(log_probs, attn, prev_attn_peak, sequence_scores):
    prev2 = jnp.broadcast_to(prev_attn_peak[:, None], (BATCH * BEAM, BEAM))
    seq2 = jnp.broadcast_to(sequence_scores[:, None], (BATCH * BEAM, BEAM))
    out3 = jax.ShapeDtypeStruct((BATCH, 1, BEAM), jnp.float32)
    out3i = jax.ShapeDtypeStruct((BATCH, 1, BEAM), jnp.int32)
    ts, tok, pred = pl.pallas_call(
        _step_kernel,
        grid=(BATCH,),
        in_specs=[
            pl.BlockSpec((BEAM, BEAM), lambda b: (b, 0)),
            pl.BlockSpec((BEAM, BEAM), lambda b: (b, 0)),
            pl.BlockSpec((BEAM, VOCAB), lambda b: (b, 0)),
            pl.BlockSpec((BEAM, ENC_LEN), lambda b: (b, 0)),
        ],
        out_specs=[
            pl.BlockSpec((1, 1, BEAM), lambda b: (b, 0, 0)),
            pl.BlockSpec((1, 1, BEAM), lambda b: (b, 0, 0)),
            pl.BlockSpec((1, 1, BEAM), lambda b: (b, 0, 0)),
        ],
        out_shape=[out3, out3i, out3i],
    )(prev2, seq2, log_probs, attn)
    return ts.reshape(BATCH, BEAM), tok.reshape(BATCH, BEAM), pred.reshape(BATCH, BEAM)


# speedup vs baseline: 1.7482x; 1.0984x over previous
import functools

import jax
import jax.numpy as jnp
from jax import lax
from jax.experimental import pallas as pl
from jax.experimental.pallas import tpu as pltpu
from jax.experimental.pallas import tpu_sc as plsc

BATCH = 16
BEAM = 8
VOCAB = 32768
ENC_LEN = 2048
EOS_INDEX = 32767
MAX_ATTN_SHIFT = 60.0
EOS_THRESHOLD = 1.5
MINUS_INF = -1e20
NEG_INF = float("-inf")
BIG_I32 = 2**30
C = 512
NB = VOCAB // C
ROWS_PER_TILE = 4
NCAND = BEAM * NB + 16
NC2 = NCAND // 16
NFIN = BEAM * C + 16
NF2 = NFIN // 16


def _sc_body(lp_hbm, attn_hbm, prev_hbm, seq_hbm,
             ts_hbm, tok_hbm, pred_hbm,
             prev_v, seq_v, attn_buf, lp_buf, bm_buf, misc_buf,
             shared_bm, shared_misc,
             bmflat, idflat, condbuf, selv_buf, seli_buf, gbuf,
             candv, candi, ob_ts, ob_tok, ob_pred):
    cid = lax.axis_index("c")
    sid = lax.axis_index("s")
    lane = lax.iota(jnp.int32, 16)
    f32 = jnp.float32

    pltpu.sync_copy(prev_hbm, prev_v.at[pl.ds(0, BATCH * BEAM)])
    pltpu.sync_copy(seq_hbm, seq_v.at[pl.ds(0, BATCH * BEAM)])

    row0 = cid * 64 + sid * ROWS_PER_TILE
    misc = jnp.full((16,), f32(MINUS_INF))
    misc = jnp.where(lane >= 4, f32(0.0), misc)
    misc_buf[...] = misc

    for ri in range(ROWS_PER_TILE):
        r = row0 + ri
        pltpu.sync_copy(attn_hbm.at[pl.ds(pl.multiple_of(r * ENC_LEN, ENC_LEN), ENC_LEN)],
                        attn_buf)

        def attn_step(i, carry):
            mx, ix = carry
            v = attn_buf[pl.ds(i * 16, 16)]
            upd = v > mx
            return (jnp.where(upd, v, mx),
                    jnp.where(upd, i * 16 + lane, ix))

        mx0 = jnp.full((16,), f32(NEG_INF))
        ix0 = jnp.zeros((16,), jnp.int32)
        mx, ix = lax.fori_loop(0, ENC_LEN // 16, attn_step, (mx0, ix0))
        am = jnp.max(mx)
        peak = jnp.min(jnp.where(mx == am, ix, BIG_I32))
        cond = peak.astype(f32) < prev_v[pl.ds(r, 16)][0] + f32(MAX_ATTN_SHIFT)

        for g in range(NB // 16):
            bm_buf[pl.ds(g * 16, 16)] = jnp.full((16,), f32(MINUS_INF))
        mv = misc_buf[...]
        misc_buf[...] = jnp.where(lane == 4 + ri,
                                  jnp.where(cond, f32(1.0), f32(0.0)), mv)

        @pl.when(cond)
        def _():
            pltpu.sync_copy(lp_hbm.at[pl.ds(pl.multiple_of(r * VOCAB, VOCAB), VOCAB)],
                            lp_buf)
            tail = lp_buf[pl.ds(VOCAB - 16, 16)]
            eos_lp = tail[15]
            lp_buf[pl.ds(VOCAB - 16, 16)] = jnp.where(lane == 15, f32(NEG_INF), tail)
            seq_r = seq_v[pl.ds(r, 16)][0]

            def blk_max(j, acc16):
                def step(t, vm):
                    return jnp.maximum(vm, lp_buf[pl.ds(j * C + t * 16, 16)])
                vm = lax.fori_loop(0, C // 16, step, jnp.full((16,), f32(NEG_INF)))
                bmj = jnp.max(vm)
                return jnp.where(lane == jnp.bitwise_and(j, 15), bmj, acc16)

            m_nz = f32(NEG_INF)
            for g in range(NB // 16):
                acc = lax.fori_loop(g * 16, g * 16 + 16, blk_max,
                                    jnp.full((16,), f32(NEG_INF)))
                bm_buf[pl.ds(g * 16, 16)] = seq_r + acc
                m_nz = jnp.maximum(m_nz, jnp.max(acc))

            eosc = jnp.where(eos_lp >= f32(EOS_THRESHOLD) * m_nz,
                             seq_r + eos_lp, f32(MINUS_INF))
            mv2 = misc_buf[...]
            misc_buf[...] = jnp.where(lane == ri, eosc, mv2)

        pltpu.sync_copy(bm_buf, shared_bm.at[pl.ds(
            pl.multiple_of((sid * ROWS_PER_TILE + ri) * NB, NB), NB)])

    pltpu.sync_copy(misc_buf, shared_misc.at[pl.ds(pl.multiple_of(sid * 16, 16), 16)])
    plsc.subcore_barrier()

    @pl.when(sid < 8)
    def _():
        b = cid * 8 + sid
        pltpu.sync_copy(
            shared_bm.at[pl.ds(pl.multiple_of(sid * (8 * NB), 8 * NB), 8 * NB)],
            bmflat.at[pl.ds(0, 8 * NB)])
        pltpu.sync_copy(shared_misc.at[pl.ds(pl.multiple_of(sid * 32, 32), 32)],
                        attn_buf.at[pl.ds(0, 32)])
        mrow = jnp.bitwise_and(jnp.right_shift(lane, 2), 1) * 16
        eosg = plsc.load_gather(attn_buf, [mrow + jnp.bitwise_and(lane, 3)])
        eos8 = jnp.where(lane < 8, eosg, f32(NEG_INF))
        bmflat[pl.ds(BEAM * NB, 16)] = eos8
        condg = plsc.load_gather(attn_buf, [mrow + 4 + jnp.bitwise_and(lane, 3)])
        condbuf[pl.ds(0, 16)] = condg
        condbuf[pl.ds(16, 16)] = jnp.zeros((16,), f32)

        def mkid(i, _):
            pos = i * 16 + lane
            jreg = jnp.right_shift(pos, 6)
            kreg = jnp.bitwise_and(pos, 63)
            rid = jreg * VOCAB + kreg * C
            eid = (pos - BEAM * NB) * VOCAB + EOS_INDEX
            idv = jnp.where(pos < BEAM * NB, rid,
                            jnp.where(pos < BEAM * NB + 8, eid, BIG_I32))
            idflat[pl.ds(i * 16, 16)] = idv
            return 0

        lax.fori_loop(0, NC2, mkid, 0)

        selv = jnp.zeros((16,), f32)
        seli = jnp.zeros((16,), jnp.int32)
        for k in range(BEAM):
            def scan2(i, carry):
                bv, bi = carry
                v = bmflat[pl.ds(i * 16, 16)]
                idv = idflat[pl.ds(i * 16, 16)]
                upd = (v > bv) | ((v == bv) & (idv < bi))
                return jnp.where(upd, v, bv), jnp.where(upd, idv, bi)

            bv, bi = lax.fori_loop(0, NC2, scan2,
                                   (jnp.full((16,), f32(NEG_INF)),
                                    jnp.full((16,), BIG_I32)))
            m = jnp.max(bv)
            win = jnp.min(jnp.where(bv == m, bi, BIG_I32))
            selv = jnp.where(lane == k, m, selv)
            seli = jnp.where(lane == k, win, seli)
            off = jnp.bitwise_and(win, VOCAB - 1)
            jw = jnp.right_shift(win, 15)
            pos = jnp.where(off == EOS_INDEX, BEAM * NB + jw,
                            jw * NB + jnp.right_shift(off, 9))
            al = jnp.bitwise_and(pos, ~15)
            q = bmflat[pl.ds(al, 16)]
            bmflat[pl.ds(al, 16)] = jnp.where(lane == jnp.bitwise_and(pos, 15),
                                              f32(NEG_INF), q)
        selv_buf[...] = selv
        seli_buf[...] = seli

        evs = jnp.full((16,), f32(NEG_INF))
        evi = jnp.full((16,), BIG_I32)
        selv_all = selv_buf[...]
        seli_all = seli_buf[...]
        for k in range(BEAM):
            win = seli_all[k]
            wval = selv_all[k]
            jw = jnp.right_shift(win, 15)
            off = jnp.bitwise_and(win, VOCAB - 1)
            iseos = off == EOS_INDEX
            start = jnp.left_shift(jnp.right_shift(off, 9), 9)
            flat = pl.multiple_of((b * BEAM + jw) * VOCAB + start, C)
            pltpu.sync_copy(lp_hbm.at[pl.ds(flat, C)], gbuf)
            ebf = jnp.where(iseos, f32(1.0), f32(0.0))
            cj = condbuf[pl.ds(jw, 16)][0]
            sj = seq_v[pl.ds(b * BEAM + jw, 16)][0]

            def resc(t, _):
                v = gbuf[pl.ds(t * 16, 16)]
                posv = start + t * 16 + lane
                sc = cj * (sj + v) - (f32(1.0) - cj) * f32(1e20)
                sc = jnp.where(posv == EOS_INDEX, f32(NEG_INF), sc)
                sc = sc - ebf * f32(2e38)
                candv[pl.ds(k * C + t * 16, 16)] = sc
                candi[pl.ds(k * C + t * 16, 16)] = jw * VOCAB + posv
                return 0

            lax.fori_loop(0, C // 16, resc, 0)
            evs = jnp.where(lane == k,
                            ebf * wval - (f32(1.0) - ebf) * f32(2e38), evs)
            evi = jnp.where(lane == k, jnp.where(iseos, win, BIG_I32), evi)
        candv[pl.ds(BEAM * C, 16)] = evs
        candi[pl.ds(BEAM * C, 16)] = evi

        ots = jnp.zeros((16,), f32)
        otok = jnp.zeros((16,), jnp.int32)
        opred = jnp.zeros((16,), jnp.int32)
        pv = f32(float("inf"))
        pi = jnp.int32(-1)
        for k in range(BEAM):
            def scan3(i, carry):
                bv, bi = carry
                v = candv[pl.ds(i * 16, 16)]
                idv = candi[pl.ds(i * 16, 16)]
                elig = (v < pv) | ((v == pv) & (idv > pi))
                upd = elig & ((v > bv) | ((v == bv) & (idv < bi)))
                return jnp.where(upd, v, bv), jnp.where(upd, idv, bi)

            bv, bi = lax.fori_loop(0, NF2, scan3,
                                   (jnp.full((16,), f32(NEG_INF)),
                                    jnp.full((16,), BIG_I32)))
            m = jnp.max(bv)
            wi = jnp.min(jnp.where(bv == m, bi, BIG_I32))
            ots = jnp.where(lane == k, m, ots)
            otok = jnp.where(lane == k, jnp.bitwise_and(wi, VOCAB - 1), otok)
            opred = jnp.where(lane == k, jnp.right_shift(wi, 15) + b * BEAM, opred)
            pv = m
            pi = wi

        ob_ts[...] = ots
        ob_tok[...] = otok
        ob_pred[...] = opred
        ob = pl.ds(pl.multiple_of(b * 16, 16), 16)
        pltpu.sync_copy(ob_ts, ts_hbm.at[ob])
        pltpu.sync_copy(ob_tok, tok_hbm.at[ob])
        pltpu.sync_copy(ob_pred, pred_hbm.at[ob])


@jax.jit
def kernel(log_probs, attn, prev_attn_peak, sequence_scores):
    mesh = plsc.VectorSubcoreMesh(core_axis_name="c", subcore_axis_name="s")
    run = pl.kernel(
        _sc_body,
        out_type=[
            jax.ShapeDtypeStruct((BATCH * 16,), jnp.float32),
            jax.ShapeDtypeStruct((BATCH * 16,), jnp.int32),
            jax.ShapeDtypeStruct((BATCH * 16,), jnp.int32),
        ],
        mesh=mesh,
        compiler_params=pltpu.CompilerParams(needs_layout_passes=False),
        scratch_types=[
            pltpu.VMEM((BATCH * BEAM + 16,), jnp.float32),
            pltpu.VMEM((BATCH * BEAM + 16,), jnp.float32),
            pltpu.VMEM((ENC_LEN,), jnp.float32),
            pltpu.VMEM((VOCAB,), jnp.float32),
            pltpu.VMEM((NB,), jnp.float32),
            pltpu.VMEM((16,), jnp.float32),
            pltpu.VMEM_SHARED((64 * NB,), jnp.float32),
            pltpu.VMEM_SHARED((256,), jnp.float32),
            pltpu.VMEM((NCAND,), jnp.float32),
            pltpu.VMEM((NCAND,), jnp.int32),
            pltpu.VMEM((32,), jnp.float32),
            pltpu.VMEM((16,), jnp.float32),
            pltpu.VMEM((16,), jnp.int32),
            pltpu.VMEM((C,), jnp.float32),
            pltpu.VMEM((NFIN,), jnp.float32),
            pltpu.VMEM((NFIN,), jnp.int32),
            pltpu.VMEM((16,), jnp.float32),
            pltpu.VMEM((16,), jnp.int32),
            pltpu.VMEM((16,), jnp.int32),
        ],
    )
    ts, tok, pred = run(log_probs.reshape(-1), attn.reshape(-1),
                        prev_attn_peak, sequence_scores)
    return (ts.reshape(BATCH, 16)[:, :BEAM],
            tok.reshape(BATCH, 16)[:, :BEAM],
            pred.reshape(BATCH, 16)[:, :BEAM])


# speedup vs baseline: 3.1753x; 1.8163x over previous
import jax
import jax.numpy as jnp
from jax import lax
from jax.experimental import pallas as pl
from jax.experimental.pallas import tpu as pltpu
from jax.experimental.pallas import tpu_sc as plsc

BATCH = 16
BEAM = 8
VOCAB = 32768
ENC_LEN = 2048
EOS_INDEX = 32767
MAX_ATTN_SHIFT = 60.0
EOS_THRESHOLD = 1.5
MINUS_INF = -1e20
NEG_INF = float("-inf")
BIG_I32 = 2**30
C = 512
NB = VOCAB // C
ROWS_PER_TILE = 4
NCAND = BEAM * NB + 16
NC2 = NCAND // 16
NFIN = BEAM * C + 16
NF2 = NFIN // 16


def _lexmax(v0, i0, v1, i1):
    upd = (v1 > v0) | ((v1 == v0) & (i1 < i0))
    return jnp.where(upd, v1, v0), jnp.where(upd, i1, i0)


def _sc_body(lp_hbm, attn_hbm, prev_hbm, seq_hbm,
             ts_hbm, tok_hbm, pred_hbm,
             prev_v, seq_v, attn_buf, lp_buf, bm_buf, misc_buf,
             shared_bm, shared_misc,
             bmflat, idflat, condbuf, selv_buf, seli_buf, gbuf,
             candv, candi, ob_ts, ob_tok, ob_pred, sem):
    cid = lax.axis_index("c")
    sid = lax.axis_index("s")
    lane = lax.iota(jnp.int32, 16)
    f32 = jnp.float32

    pltpu.sync_copy(prev_hbm, prev_v.at[pl.ds(0, BATCH * BEAM)])
    pltpu.sync_copy(seq_hbm, seq_v.at[pl.ds(0, BATCH * BEAM)])

    row0 = cid * 64 + sid * ROWS_PER_TILE
    pltpu.sync_copy(attn_hbm.at[pl.ds(pl.multiple_of(row0, ROWS_PER_TILE),
                                      ROWS_PER_TILE)], attn_buf)
    misc = jnp.full((16,), f32(MINUS_INF))
    misc = jnp.where(lane >= 4, f32(0.0), misc)

    for ri in range(ROWS_PER_TILE):
        r = row0 + ri

        def attn_step(i, carry):
            m0, x0, m1, x1, m2, x2, m3, x3 = carry
            base = i * 64
            v0 = attn_buf[ri, pl.ds(base, 16)]
            v1 = attn_buf[ri, pl.ds(base + 16, 16)]
            v2 = attn_buf[ri, pl.ds(base + 32, 16)]
            v3 = attn_buf[ri, pl.ds(base + 48, 16)]
            u0 = v0 > m0
            u1 = v1 > m1
            u2 = v2 > m2
            u3 = v3 > m3
            return (jnp.where(u0, v0, m0), jnp.where(u0, base + lane, x0),
                    jnp.where(u1, v1, m1), jnp.where(u1, base + 16 + lane, x1),
                    jnp.where(u2, v2, m2), jnp.where(u2, base + 32 + lane, x2),
                    jnp.where(u3, v3, m3), jnp.where(u3, base + 48 + lane, x3))

        mninf = jnp.full((16,), f32(NEG_INF))
        zi = jnp.zeros((16,), jnp.int32)
        m0, x0, m1, x1, m2, x2, m3, x3 = lax.fori_loop(
            0, ENC_LEN // 64, attn_step, (mninf, zi, mninf, zi, mninf, zi, mninf, zi))
        ma, xa = _lexmax(m0, x0, m1, x1)
        mb, xb = _lexmax(m2, x2, m3, x3)
        mx, ix = _lexmax(ma, xa, mb, xb)
        am = jnp.max(mx)
        peak = jnp.min(jnp.where(mx == am, ix, BIG_I32))
        cond = peak.astype(f32) < prev_v[pl.ds(r, 16)][0] + f32(MAX_ATTN_SHIFT)

        for g in range(NB // 16):
            bm_buf[pl.ds(ri * NB + g * 16, 16)] = jnp.full((16,), f32(MINUS_INF))
        misc = jnp.where(lane == 4 + ri, jnp.where(cond, f32(1.0), f32(0.0)), misc)
        misc_buf[...] = misc

        @pl.when(cond)
        def _():
            pltpu.sync_copy(lp_hbm.at[r], lp_buf)
            tail = lp_buf[pl.ds(VOCAB - 16, 16)]
            eos_lp = tail[15]
            lp_buf[pl.ds(VOCAB - 16, 16)] = jnp.where(lane == 15, f32(NEG_INF), tail)
            seq_r = seq_v[pl.ds(r, 16)][0]

            def blk_max(j, acc16):
                def step(t, carry):
                    a0, a1, a2, a3 = carry
                    base = j * C + t * 64
                    return (jnp.maximum(a0, lp_buf[pl.ds(base, 16)]),
                            jnp.maximum(a1, lp_buf[pl.ds(base + 16, 16)]),
                            jnp.maximum(a2, lp_buf[pl.ds(base + 32, 16)]),
                            jnp.maximum(a3, lp_buf[pl.ds(base + 48, 16)]))

                a0, a1, a2, a3 = lax.fori_loop(
                    0, C // 64, step, (mninf, mninf, mninf, mninf))
                vm = jnp.maximum(jnp.maximum(a0, a1), jnp.maximum(a2, a3))
                bmj = jnp.max(vm)
                return jnp.where(lane == jnp.bitwise_and(j, 15), bmj, acc16)

            m_nz = f32(NEG_INF)
            for g in range(NB // 16):
                acc = lax.fori_loop(g * 16, g * 16 + 16, blk_max,
                                    jnp.full((16,), f32(NEG_INF)))
                bm_buf[pl.ds(ri * NB + g * 16, 16)] = seq_r + acc
                m_nz = jnp.maximum(m_nz, jnp.max(acc))

            eosc = jnp.where(eos_lp >= f32(EOS_THRESHOLD) * m_nz,
                             seq_r + eos_lp, f32(MINUS_INF))
            mv2 = misc_buf[...]
            misc_buf[...] = jnp.where(lane == ri, eosc, mv2)

    pltpu.sync_copy(bm_buf, shared_bm.at[pl.ds(
        pl.multiple_of(sid * (ROWS_PER_TILE * NB), ROWS_PER_TILE * NB),
        ROWS_PER_TILE * NB)])
    pltpu.sync_copy(misc_buf, shared_misc.at[pl.ds(pl.multiple_of(sid * 16, 16), 16)])
    plsc.subcore_barrier()

    @pl.when(sid < 8)
    def _():
        b = cid * 8 + sid
        pltpu.sync_copy(
            shared_bm.at[pl.ds(pl.multiple_of(sid * (8 * NB), 8 * NB), 8 * NB)],
            bmflat.at[pl.ds(0, 8 * NB)])
        pltpu.sync_copy(shared_misc.at[pl.ds(pl.multiple_of(sid * 32, 32), 32)],
                        condbuf.at[pl.ds(16, 32)])
        mrow = jnp.bitwise_and(jnp.right_shift(lane, 2), 1) * 16
        eosg = plsc.load_gather(condbuf, [16 + mrow + jnp.bitwise_and(lane, 3)])
        eos8 = jnp.where(lane < 8, eosg, f32(NEG_INF))
        bmflat[pl.ds(BEAM * NB, 16)] = eos8
        condg = plsc.load_gather(condbuf, [16 + mrow + 4 + jnp.bitwise_and(lane, 3)])
        condbuf[pl.ds(0, 16)] = condg

        def mkid(i, _):
            pos = i * 16 + lane
            jreg = jnp.right_shift(pos, 6)
            kreg = jnp.bitwise_and(pos, 63)
            rid = jreg * VOCAB + kreg * C
            eid = (pos - BEAM * NB) * VOCAB + EOS_INDEX
            idv = jnp.where(pos < BEAM * NB, rid,
                            jnp.where(pos < BEAM * NB + 8, eid, BIG_I32))
            idflat[pl.ds(i * 16, 16)] = idv
            return 0

        lax.fori_loop(0, NC2, mkid, 0)

        selv = jnp.zeros((16,), f32)
        seli = jnp.zeros((16,), jnp.int32)
        for k in range(BEAM):
            def scan2(i, carry):
                bv, bi = carry
                v = bmflat[pl.ds(i * 16, 16)]
                idv = idflat[pl.ds(i * 16, 16)]
                upd = (v > bv) | ((v == bv) & (idv < bi))
                return jnp.where(upd, v, bv), jnp.where(upd, idv, bi)

            bv, bi = lax.fori_loop(0, NC2, scan2,
                                   (jnp.full((16,), f32(NEG_INF)),
                                    jnp.full((16,), BIG_I32)))
            m = jnp.max(bv)
            win = jnp.min(jnp.where(bv == m, bi, BIG_I32))
            selv = jnp.where(lane == k, m, selv)
            seli = jnp.where(lane == k, win, seli)
            off = jnp.bitwise_and(win, VOCAB - 1)
            jw = jnp.right_shift(win, 15)
            pos = jnp.where(off == EOS_INDEX, BEAM * NB + jw,
                            jw * NB + jnp.right_shift(off, 9))
            al = jnp.bitwise_and(pos, ~15)
            q = bmflat[pl.ds(al, 16)]
            bmflat[pl.ds(al, 16)] = jnp.where(lane == jnp.bitwise_and(pos, 15),
                                              f32(NEG_INF), q)
        selv_buf[...] = selv
        seli_buf[...] = seli

        selv_all = selv_buf[...]
        seli_all = seli_buf[...]
        copies = []
        for k in range(BEAM):
            win = seli_all[k]
            jw = jnp.right_shift(win, 15)
            off = jnp.bitwise_and(win, VOCAB - 1)
            start = jnp.left_shift(jnp.right_shift(off, 9), 9)
            copies.append(pltpu.async_copy(
                lp_hbm.at[b * BEAM + jw,
                          pl.ds(pl.multiple_of(start, C), C)],
                gbuf.at[pl.ds(k * C, C)], sem))
        for cp in copies:
            cp.wait()

        evs = jnp.full((16,), f32(NEG_INF))
        evi = jnp.full((16,), BIG_I32)
        for k in range(BEAM):
            win = seli_all[k]
            wval = selv_all[k]
            jw = jnp.right_shift(win, 15)
            off = jnp.bitwise_and(win, VOCAB - 1)
            iseos = off == EOS_INDEX
            start = jnp.left_shift(jnp.right_shift(off, 9), 9)
            ebf = jnp.where(iseos, f32(1.0), f32(0.0))
            cj = condbuf[pl.ds(jw, 16)][0]
            sj = seq_v[pl.ds(b * BEAM + jw, 16)][0]

            def resc(t, _):
                v = gbuf[pl.ds(k * C + t * 16, 16)]
                posv = start + t * 16 + lane
                sc = cj * (sj + v) - (f32(1.0) - cj) * f32(1e20)
                sc = jnp.where(posv == EOS_INDEX, f32(NEG_INF), sc)
                sc = sc - ebf * f32(2e38)
                candv[pl.ds(k * C + t * 16, 16)] = sc
                candi[pl.ds(k * C + t * 16, 16)] = jw * VOCAB + posv
                return 0

            lax.fori_loop(0, C // 16, resc, 0)
            evs = jnp.where(lane == k,
                            ebf * wval - (f32(1.0) - ebf) * f32(2e38), evs)
            evi = jnp.where(lane == k, jnp.where(iseos, win, BIG_I32), evi)
        candv[pl.ds(BEAM * C, 16)] = evs
        candi[pl.ds(BEAM * C, 16)] = evi

        ots = jnp.zeros((16,), f32)
        otok = jnp.zeros((16,), jnp.int32)
        opred = jnp.zeros((16,), jnp.int32)
        pv = f32(float("inf"))
        pi = jnp.int32(-1)
        for k in range(BEAM):
            def scan3(i, carry):
                bv, bi = carry
                v = candv[pl.ds(i * 16, 16)]
                idv = candi[pl.ds(i * 16, 16)]
                elig = (v < pv) | ((v == pv) & (idv > pi))
                upd = elig & ((v > bv) | ((v == bv) & (idv < bi)))
                return jnp.where(upd, v, bv), jnp.where(upd, idv, bi)

            bv, bi = lax.fori_loop(0, NF2, scan3,
                                   (jnp.full((16,), f32(NEG_INF)),
                                    jnp.full((16,), BIG_I32)))
            m = jnp.max(bv)
            wi = jnp.min(jnp.where(bv == m, bi, BIG_I32))
            ots = jnp.where(lane == k, m, ots)
            otok = jnp.where(lane == k, jnp.bitwise_and(wi, VOCAB - 1), otok)
            opred = jnp.where(lane == k, jnp.right_shift(wi, 15) + b * BEAM, opred)
            pv = m
            pi = wi

        ob_ts[...] = ots
        ob_tok[...] = otok
        ob_pred[...] = opred
        ob = pl.ds(pl.multiple_of(b * 16, 16), 16)
        pltpu.sync_copy(ob_ts, ts_hbm.at[ob])
        pltpu.sync_copy(ob_tok, tok_hbm.at[ob])
        pltpu.sync_copy(ob_pred, pred_hbm.at[ob])


@jax.jit
def kernel(log_probs, attn, prev_attn_peak, sequence_scores):
    mesh = plsc.VectorSubcoreMesh(core_axis_name="c", subcore_axis_name="s")
    run = pl.kernel(
        _sc_body,
        out_type=[
            jax.ShapeDtypeStruct((BATCH * 16,), jnp.float32),
            jax.ShapeDtypeStruct((BATCH * 16,), jnp.int32),
            jax.ShapeDtypeStruct((BATCH * 16,), jnp.int32),
        ],
        mesh=mesh,
        compiler_params=pltpu.CompilerParams(needs_layout_passes=False),
        scratch_types=[
            pltpu.VMEM((BATCH * BEAM + 16,), jnp.float32),
            pltpu.VMEM((BATCH * BEAM + 16,), jnp.float32),
            pltpu.VMEM((ROWS_PER_TILE, ENC_LEN), jnp.float32),
            pltpu.VMEM((VOCAB,), jnp.float32),
            pltpu.VMEM((ROWS_PER_TILE * NB,), jnp.float32),
            pltpu.VMEM((16,), jnp.float32),
            pltpu.VMEM_SHARED((64 * NB,), jnp.float32),
            pltpu.VMEM_SHARED((256,), jnp.float32),
            pltpu.VMEM((NCAND,), jnp.float32),
            pltpu.VMEM((NCAND,), jnp.int32),
            pltpu.VMEM((48,), jnp.float32),
            pltpu.VMEM((16,), jnp.float32),
            pltpu.VMEM((16,), jnp.int32),
            pltpu.VMEM((BEAM * C,), jnp.float32),
            pltpu.VMEM((NFIN,), jnp.float32),
            pltpu.VMEM((NFIN,), jnp.int32),
            pltpu.VMEM((16,), jnp.float32),
            pltpu.VMEM((16,), jnp.int32),
            pltpu.VMEM((16,), jnp.int32),
            pltpu.SemaphoreType.DMA,
        ],
    )
    ts, tok, pred = run(log_probs, attn, prev_attn_peak, sequence_scores)
    return (ts.reshape(BATCH, 16)[:, :BEAM],
            tok.reshape(BATCH, 16)[:, :BEAM],
            pred.reshape(BATCH, 16)[:, :BEAM])


# speedup vs baseline: 3.5493x; 1.1178x over previous
import jax
import jax.numpy as jnp
from jax import lax
from jax.experimental import pallas as pl
from jax.experimental.pallas import tpu as pltpu
from jax.experimental.pallas import tpu_sc as plsc

BATCH = 16
BEAM = 8
VOCAB = 32768
ENC_LEN = 2048
EOS_INDEX = 32767
MAX_ATTN_SHIFT = 60.0
EOS_THRESHOLD = 1.5
MINUS_INF = -1e20
NEG_INF = float("-inf")
BIG_I32 = 2**30
C = 512
NB = VOCAB // C
ROWS_PER_TILE = 4
NCAND = BEAM * NB + 16
NC2 = NCAND // 16
CHUNK = 256
NCH = (BEAM * C) // CHUNK + 1
NFIN = NCH * CHUNK


def _lexmax(v0, i0, v1, i1):
    upd = (v1 > v0) | ((v1 == v0) & (i1 < i0))
    return jnp.where(upd, v1, v0), jnp.where(upd, i1, i0)


def _sc_body(lp_hbm, attn_hbm, prev_hbm, seq_hbm,
             ts_hbm, tok_hbm, pred_hbm,
             prev_v, seq_v, attn_buf, lp_buf, bm_buf, misc_buf,
             shared_bm, shared_misc,
             bmflat, idflat, condbuf, selv_buf, seli_buf, gbuf,
             candv, candi, sumv, sumi, ob_ts, ob_tok, ob_pred,
             sem, sem2):
    cid = lax.axis_index("c")
    sid = lax.axis_index("s")
    lane = lax.iota(jnp.int32, 16)
    f32 = jnp.float32

    pltpu.sync_copy(prev_hbm, prev_v.at[pl.ds(0, BATCH * BEAM)])
    pltpu.sync_copy(seq_hbm, seq_v.at[pl.ds(0, BATCH * BEAM)])

    row0 = cid * 64 + sid * ROWS_PER_TILE
    pltpu.sync_copy(attn_hbm.at[pl.ds(pl.multiple_of(row0, ROWS_PER_TILE),
                                      ROWS_PER_TILE)], attn_buf)
    misc = jnp.full((16,), f32(MINUS_INF))
    misc = jnp.where(lane >= 4, f32(0.0), misc)
    mninf = jnp.full((16,), f32(NEG_INF))
    zi = jnp.zeros((16,), jnp.int32)

    for ri in range(ROWS_PER_TILE):
        r = row0 + ri

        def attn_step(i, carry):
            m0, x0, m1, x1, m2, x2, m3, x3 = carry
            base = i * 64
            v0 = attn_buf[ri, pl.ds(base, 16)]
            v1 = attn_buf[ri, pl.ds(base + 16, 16)]
            v2 = attn_buf[ri, pl.ds(base + 32, 16)]
            v3 = attn_buf[ri, pl.ds(base + 48, 16)]
            u0 = v0 > m0
            u1 = v1 > m1
            u2 = v2 > m2
            u3 = v3 > m3
            return (jnp.where(u0, v0, m0), jnp.where(u0, base + lane, x0),
                    jnp.where(u1, v1, m1), jnp.where(u1, base + 16 + lane, x1),
                    jnp.where(u2, v2, m2), jnp.where(u2, base + 32 + lane, x2),
                    jnp.where(u3, v3, m3), jnp.where(u3, base + 48 + lane, x3))

        m0, x0, m1, x1, m2, x2, m3, x3 = lax.fori_loop(
            0, ENC_LEN // 64, attn_step, (mninf, zi, mninf, zi, mninf, zi, mninf, zi))
        ma, xa = _lexmax(m0, x0, m1, x1)
        mb, xb = _lexmax(m2, x2, m3, x3)
        mx, ix = _lexmax(ma, xa, mb, xb)
        am = jnp.max(mx)
        peak = jnp.min(jnp.where(mx == am, ix, BIG_I32))
        cond = peak.astype(f32) < prev_v[pl.ds(r, 16)][0] + f32(MAX_ATTN_SHIFT)

        for g in range(NB // 16):
            bm_buf[pl.ds(ri * NB + g * 16, 16)] = jnp.full((16,), f32(MINUS_INF))
        misc = jnp.where(lane == 4 + ri, jnp.where(cond, f32(1.0), f32(0.0)), misc)
        misc_buf[...] = misc

        @pl.when(cond)
        def _():
            quarter = VOCAB // 4
            cps = [pltpu.async_copy(
                lp_hbm.at[r, pl.ds(q * quarter, quarter)],
                lp_buf.at[pl.ds(q * quarter, quarter)],
                sem if q % 2 == 0 else sem2) for q in range(2)]
            seq_r = seq_v[pl.ds(r, 16)][0]

            def blk_max(j, acc16):
                def step(t, carry):
                    a0, a1, a2, a3 = carry
                    base = j * C + t * 64
                    return (jnp.maximum(a0, lp_buf[pl.ds(base, 16)]),
                            jnp.maximum(a1, lp_buf[pl.ds(base + 16, 16)]),
                            jnp.maximum(a2, lp_buf[pl.ds(base + 32, 16)]),
                            jnp.maximum(a3, lp_buf[pl.ds(base + 48, 16)]))

                a0, a1, a2, a3 = lax.fori_loop(
                    0, C // 64, step, (mninf, mninf, mninf, mninf))
                vm = jnp.maximum(jnp.maximum(a0, a1), jnp.maximum(a2, a3))
                bmj = jnp.max(vm)
                return jnp.where(lane == jnp.bitwise_and(j, 15), bmj, acc16)

            m_nz = f32(NEG_INF)
            for g in range(NB // 16):
                cps[g].wait()
                if g < 2:
                    cps.append(pltpu.async_copy(
                        lp_hbm.at[r, pl.ds((g + 2) * quarter, quarter)],
                        lp_buf.at[pl.ds((g + 2) * quarter, quarter)],
                        sem if g % 2 == 0 else sem2))
                if g == 3:
                    tail = lp_buf[pl.ds(VOCAB - 16, 16)]
                    eos_lp = tail[15]
                    lp_buf[pl.ds(VOCAB - 16, 16)] = jnp.where(
                        lane == 15, f32(NEG_INF), tail)
                acc = lax.fori_loop(g * 16, g * 16 + 16, blk_max,
                                    jnp.full((16,), f32(NEG_INF)))
                bm_buf[pl.ds(ri * NB + g * 16, 16)] = seq_r + acc
                m_nz = jnp.maximum(m_nz, jnp.max(acc))

            eosc = jnp.where(eos_lp >= f32(EOS_THRESHOLD) * m_nz,
                             seq_r + eos_lp, f32(MINUS_INF))
            mv2 = misc_buf[...]
            misc_buf[...] = jnp.where(lane == ri, eosc, mv2)

    pltpu.sync_copy(bm_buf, shared_bm.at[pl.ds(
        pl.multiple_of(sid * (ROWS_PER_TILE * NB), ROWS_PER_TILE * NB),
        ROWS_PER_TILE * NB)])
    pltpu.sync_copy(misc_buf, shared_misc.at[pl.ds(pl.multiple_of(sid * 16, 16), 16)])
    plsc.subcore_barrier()

    @pl.when(sid < 8)
    def _():
        b = cid * 8 + sid
        pltpu.sync_copy(
            shared_bm.at[pl.ds(pl.multiple_of(sid * (8 * NB), 8 * NB), 8 * NB)],
            bmflat.at[pl.ds(0, 8 * NB)])
        pltpu.sync_copy(shared_misc.at[pl.ds(pl.multiple_of(sid * 32, 32), 32)],
                        condbuf.at[pl.ds(16, 32)])
        mrow = jnp.bitwise_and(jnp.right_shift(lane, 2), 1) * 16
        eosg = plsc.load_gather(condbuf, [16 + mrow + jnp.bitwise_and(lane, 3)])
        eos8 = jnp.where(lane < 8, eosg, f32(NEG_INF))
        bmflat[pl.ds(BEAM * NB, 16)] = eos8
        condg = plsc.load_gather(condbuf, [16 + mrow + 4 + jnp.bitwise_and(lane, 3)])
        condbuf[pl.ds(0, 16)] = condg

        def mkid(i, _):
            pos = i * 16 + lane
            jreg = jnp.right_shift(pos, 6)
            kreg = jnp.bitwise_and(pos, 63)
            rid = jreg * VOCAB + kreg * C
            eid = (pos - BEAM * NB) * VOCAB + EOS_INDEX
            idv = jnp.where(pos < BEAM * NB, rid,
                            jnp.where(pos < BEAM * NB + 8, eid, BIG_I32))
            idflat[pl.ds(i * 16, 16)] = idv
            return 0

        lax.fori_loop(0, NC2, mkid, 0)

        selv = jnp.zeros((16,), f32)
        seli = jnp.zeros((16,), jnp.int32)
        for k in range(BEAM):
            def scan2(i, carry):
                bv, bi = carry
                v = bmflat[pl.ds(i * 16, 16)]
                idv = idflat[pl.ds(i * 16, 16)]
                upd = (v > bv) | ((v == bv) & (idv < bi))
                return jnp.where(upd, v, bv), jnp.where(upd, idv, bi)

            bv, bi = lax.fori_loop(0, NC2, scan2,
                                   (jnp.full((16,), f32(NEG_INF)),
                                    jnp.full((16,), BIG_I32)))
            m = jnp.max(bv)
            win = jnp.min(jnp.where(bv == m, bi, BIG_I32))
            selv = jnp.where(lane == k, m, selv)
            seli = jnp.where(lane == k, win, seli)
            off = jnp.bitwise_and(win, VOCAB - 1)
            jw = jnp.right_shift(win, 15)
            pos = jnp.where(off == EOS_INDEX, BEAM * NB + jw,
                            jw * NB + jnp.right_shift(off, 9))
            al = jnp.bitwise_and(pos, ~15)
            q = bmflat[pl.ds(al, 16)]
            bmflat[pl.ds(al, 16)] = jnp.where(lane == jnp.bitwise_and(pos, 15),
                                              f32(NEG_INF), q)
        selv_buf[...] = selv
        seli_buf[...] = seli

        selv_all = selv_buf[...]
        seli_all = seli_buf[...]
        copies = []
        for k in range(BEAM):
            win = seli_all[k]
            jw = jnp.right_shift(win, 15)
            off = jnp.bitwise_and(win, VOCAB - 1)
            start = jnp.left_shift(jnp.right_shift(off, 9), 9)
            copies.append(pltpu.async_copy(
                lp_hbm.at[b * BEAM + jw,
                          pl.ds(pl.multiple_of(start, C), C)],
                gbuf.at[pl.ds(k * C, C)], sem))
        for cp in copies:
            cp.wait()

        sv_reg = mninf
        si_reg = jnp.full((16,), BIG_I32)
        evs = jnp.full((16,), f32(NEG_INF))
        evi = jnp.full((16,), BIG_I32)
        for k in range(BEAM):
            win = seli_all[k]
            wval = selv_all[k]
            jw = jnp.right_shift(win, 15)
            off = jnp.bitwise_and(win, VOCAB - 1)
            iseos = off == EOS_INDEX
            start = jnp.left_shift(jnp.right_shift(off, 9), 9)
            ebf = jnp.where(iseos, f32(1.0), f32(0.0))
            cj = condbuf[pl.ds(jw, 16)][0]
            sj = seq_v[pl.ds(b * BEAM + jw, 16)][0]

            for h in range(2):
                ch = 2 * k + h

                def resc(t, carry):
                    smx, six = carry
                    v = gbuf[pl.ds(k * C + h * CHUNK + t * 16, 16)]
                    posv = start + h * CHUNK + t * 16 + lane
                    sc = cj * (sj + v) - (f32(1.0) - cj) * f32(1e20)
                    sc = jnp.where(posv == EOS_INDEX, f32(NEG_INF), sc)
                    sc = sc - ebf * f32(2e38)
                    idv = jw * VOCAB + posv
                    candv[pl.ds(ch * CHUNK + t * 16, 16)] = sc
                    candi[pl.ds(ch * CHUNK + t * 16, 16)] = idv
                    return _lexmax(smx, six, sc, idv)

                smx, six = lax.fori_loop(0, CHUNK // 16, resc,
                                         (mninf, jnp.full((16,), BIG_I32)))
                chv = jnp.max(smx)
                chi = jnp.min(jnp.where(smx == chv, six, BIG_I32))
                sv_reg = jnp.where(lane == ch, chv, sv_reg)
                si_reg = jnp.where(lane == ch, chi, si_reg)

            evs = jnp.where(lane == k,
                            ebf * wval - (f32(1.0) - ebf) * f32(2e38), evs)
            evi = jnp.where(lane == k, jnp.where(iseos, win, BIG_I32), evi)

        candv[pl.ds(16 * CHUNK, 16)] = evs
        candi[pl.ds(16 * CHUNK, 16)] = evi

        def padc(i, _):
            candv[pl.ds(16 * CHUNK + 16 + i * 16, 16)] = mninf
            candi[pl.ds(16 * CHUNK + 16 + i * 16, 16)] = jnp.full((16,), BIG_I32)
            return 0

        lax.fori_loop(0, (CHUNK - 16) // 16, padc, 0)
        sumv[pl.ds(0, 16)] = sv_reg
        sumi[pl.ds(0, 16)] = si_reg
        ev = jnp.max(evs)
        ei = jnp.min(jnp.where(evs == ev, evi, BIG_I32))
        sumv[pl.ds(16, 16)] = jnp.where(lane == 0, ev, mninf)
        sumi[pl.ds(16, 16)] = jnp.where(lane == 0, ei, jnp.full((16,), BIG_I32))

        ots = jnp.zeros((16,), f32)
        otok = jnp.zeros((16,), jnp.int32)
        opred = jnp.zeros((16,), jnp.int32)
        pv = f32(float("inf"))
        pi = jnp.int32(-1)
        for k in range(BEAM):
            s0 = sumv[pl.ds(0, 16)]
            i0 = sumi[pl.ds(0, 16)]
            s1 = sumv[pl.ds(16, 16)]
            i1 = sumi[pl.ds(16, 16)]
            p0 = lane
            p1 = lane + 16
            bsv, bsi = _lexmax(s0, i0, s1, i1)
            bsp = jnp.where((s1 > s0) | ((s1 == s0) & (i1 < i0)), p1, p0)
            m = jnp.max(bsv)
            wi = jnp.min(jnp.where(bsv == m, bsi, BIG_I32))
            cw = jnp.min(jnp.where((bsv == m) & (bsi == wi), bsp, BIG_I32))
            ots = jnp.where(lane == k, m, ots)
            otok = jnp.where(lane == k, jnp.bitwise_and(wi, VOCAB - 1), otok)
            opred = jnp.where(lane == k, jnp.right_shift(wi, 15) + b * BEAM, opred)
            pv = m
            pi = wi

            cbase = cw * CHUNK

            def rescan(t, carry):
                bv2, bi2 = carry
                idxv = cbase + t * 16 + lane
                v = plsc.load_gather(candv, [idxv])
                idv = plsc.load_gather(candi, [idxv])
                elig = (v < pv) | ((v == pv) & (idv > pi))
                upd = elig & ((v > bv2) | ((v == bv2) & (idv < bi2)))
                return jnp.where(upd, v, bv2), jnp.where(upd, idv, bi2)

            bv2, bi2 = lax.fori_loop(0, CHUNK // 16, rescan,
                                     (mninf, jnp.full((16,), BIG_I32)))
            nv = jnp.max(bv2)
            ni = jnp.min(jnp.where(bv2 == nv, bi2, BIG_I32))
            hit0 = (cw < 16) & (lane == cw)
            hit1 = (cw >= 16) & (lane == cw - 16)
            q0v = sumv[pl.ds(0, 16)]
            q0i = sumi[pl.ds(0, 16)]
            sumv[pl.ds(0, 16)] = jnp.where(hit0, nv, q0v)
            sumi[pl.ds(0, 16)] = jnp.where(hit0, ni, q0i)
            q1v = sumv[pl.ds(16, 16)]
            q1i = sumi[pl.ds(16, 16)]
            sumv[pl.ds(16, 16)] = jnp.where(hit1, nv, q1v)
            sumi[pl.ds(16, 16)] = jnp.where(hit1, ni, q1i)

        ob_ts[...] = ots
        ob_tok[...] = otok
        ob_pred[...] = opred
        ob = pl.ds(pl.multiple_of(b * BEAM, BEAM), BEAM)
        pltpu.sync_copy(ob_ts.at[pl.ds(0, BEAM)], ts_hbm.at[ob])
        pltpu.sync_copy(ob_tok.at[pl.ds(0, BEAM)], tok_hbm.at[ob])
        pltpu.sync_copy(ob_pred.at[pl.ds(0, BEAM)], pred_hbm.at[ob])


@jax.jit
def kernel(log_probs, attn, prev_attn_peak, sequence_scores):
    mesh = plsc.VectorSubcoreMesh(core_axis_name="c", subcore_axis_name="s")
    run = pl.kernel(
        _sc_body,
        out_type=[
            jax.ShapeDtypeStruct((BATCH * BEAM,), jnp.float32),
            jax.ShapeDtypeStruct((BATCH * BEAM,), jnp.int32),
            jax.ShapeDtypeStruct((BATCH * BEAM,), jnp.int32),
        ],
        mesh=mesh,
        compiler_params=pltpu.CompilerParams(needs_layout_passes=False),
        scratch_types=[
            pltpu.VMEM((BATCH * BEAM + 16,), jnp.float32),
            pltpu.VMEM((BATCH * BEAM + 16,), jnp.float32),
            pltpu.VMEM((ROWS_PER_TILE, ENC_LEN), jnp.float32),
            pltpu.VMEM((VOCAB,), jnp.float32),
            pltpu.VMEM((ROWS_PER_TILE * NB,), jnp.float32),
            pltpu.VMEM((16,), jnp.float32),
            pltpu.VMEM_SHARED((64 * NB,), jnp.float32),
            pltpu.VMEM_SHARED((256,), jnp.float32),
            pltpu.VMEM((NCAND,), jnp.float32),
            pltpu.VMEM((NCAND,), jnp.int32),
            pltpu.VMEM((48,), jnp.float32),
            pltpu.VMEM((16,), jnp.float32),
            pltpu.VMEM((16,), jnp.int32),
            pltpu.VMEM((BEAM * C,), jnp.float32),
            pltpu.VMEM((NFIN,), jnp.float32),
            pltpu.VMEM((NFIN,), jnp.int32),
            pltpu.VMEM((48,), jnp.float32),
            pltpu.VMEM((48,), jnp.int32),
            pltpu.VMEM((16,), jnp.float32),
            pltpu.VMEM((16,), jnp.int32),
            pltpu.VMEM((16,), jnp.int32),
            pltpu.SemaphoreType.DMA,
            pltpu.SemaphoreType.DMA,
        ],
    )
    ts, tok, pred = run(log_probs, attn, prev_attn_peak, sequence_scores)
    return (ts.reshape(BATCH, BEAM),
            tok.reshape(BATCH, BEAM),
            pred.reshape(BATCH, BEAM))


# speedup vs baseline: 3.7039x; 1.0436x over previous
import jax
import jax.numpy as jnp
from jax import lax
from jax.experimental import pallas as pl
from jax.experimental.pallas import tpu as pltpu
from jax.experimental.pallas import tpu_sc as plsc

BATCH = 16
BEAM = 8
VOCAB = 32768
ENC_LEN = 2048
EOS_INDEX = 32767
MAX_ATTN_SHIFT = 60.0
EOS_THRESHOLD = 1.5
MINUS_INF = -1e20
NEG_INF = float("-inf")
BIG_I32 = 2**30
C = 512
NB = VOCAB // C
ROWS_PER_TILE = 4
NCAND = BEAM * NB + 16
NC2 = NCAND // 16
CHUNK = 256
NCH = (BEAM * C) // CHUNK + 1
NFIN = NCH * CHUNK


def _lexmax(v0, i0, v1, i1):
    upd = (v1 > v0) | ((v1 == v0) & (i1 < i0))
    return jnp.where(upd, v1, v0), jnp.where(upd, i1, i0)


def _sc_body(lp_hbm, attn_hbm, prev_hbm, seq_hbm,
             ts_hbm, tok_hbm, pred_hbm,
             prev_v, seq_v, attn_buf, lp_buf, bm_buf, misc_buf,
             shared_bm, shared_misc, shared_eos, mstage, condfl,
             bmflat, idflat, condbuf, selv_buf, seli_buf, gbuf,
             candv, candi, sumv, sumi, ob_ts, ob_tok, ob_pred,
             sem, sem2):
    cid = lax.axis_index("c")
    sid = lax.axis_index("s")
    lane = lax.iota(jnp.int32, 16)
    f32 = jnp.float32

    pltpu.sync_copy(prev_hbm, prev_v.at[pl.ds(0, BATCH * BEAM)])
    pltpu.sync_copy(seq_hbm, seq_v.at[pl.ds(0, BATCH * BEAM)])

    row0 = cid * 64 + sid * ROWS_PER_TILE
    pltpu.sync_copy(attn_hbm.at[pl.ds(pl.multiple_of(row0, ROWS_PER_TILE),
                                      ROWS_PER_TILE)], attn_buf)
    misc = jnp.full((16,), f32(MINUS_INF))
    misc = jnp.where(lane >= 4, f32(0.0), misc)
    mninf = jnp.full((16,), f32(NEG_INF))
    zi = jnp.zeros((16,), jnp.int32)

    for ri in range(ROWS_PER_TILE):
        r = row0 + ri

        def attn_step(i, carry):
            m0, x0, m1, x1, m2, x2, m3, x3 = carry
            base = i * 64
            v0 = attn_buf[ri, pl.ds(base, 16)]
            v1 = attn_buf[ri, pl.ds(base + 16, 16)]
            v2 = attn_buf[ri, pl.ds(base + 32, 16)]
            v3 = attn_buf[ri, pl.ds(base + 48, 16)]
            u0 = v0 > m0
            u1 = v1 > m1
            u2 = v2 > m2
            u3 = v3 > m3
            return (jnp.where(u0, v0, m0), jnp.where(u0, base + lane, x0),
                    jnp.where(u1, v1, m1), jnp.where(u1, base + 16 + lane, x1),
                    jnp.where(u2, v2, m2), jnp.where(u2, base + 32 + lane, x2),
                    jnp.where(u3, v3, m3), jnp.where(u3, base + 48 + lane, x3))

        m0, x0, m1, x1, m2, x2, m3, x3 = lax.fori_loop(
            0, ENC_LEN // 64, attn_step, (mninf, zi, mninf, zi, mninf, zi, mninf, zi))
        ma, xa = _lexmax(m0, x0, m1, x1)
        mb, xb = _lexmax(m2, x2, m3, x3)
        mx, ix = _lexmax(ma, xa, mb, xb)
        am = jnp.max(mx)
        peak = jnp.min(jnp.where(mx == am, ix, BIG_I32))
        cond = peak.astype(f32) < prev_v[pl.ds(r, 16)][0] + f32(MAX_ATTN_SHIFT)
        misc = jnp.where(lane == 4 + ri, jnp.where(cond, f32(1.0), f32(0.0)), misc)

    misc_buf[...] = misc
    for g in range(ROWS_PER_TILE * NB // 16):
        bm_buf[pl.ds(g * 16, 16)] = jnp.full((16,), f32(MINUS_INF))
    pltpu.sync_copy(bm_buf, shared_bm.at[pl.ds(
        pl.multiple_of(sid * (ROWS_PER_TILE * NB), ROWS_PER_TILE * NB),
        ROWS_PER_TILE * NB)])
    pltpu.sync_copy(bm_buf.at[pl.ds(0, 64)], shared_eos.at[pl.ds(
        pl.multiple_of(sid * 64, 64), 64)])
    pltpu.sync_copy(misc_buf, shared_misc.at[pl.ds(pl.multiple_of(sid * 16, 16), 16)])
    plsc.subcore_barrier()

    pltpu.sync_copy(shared_misc, mstage)
    for g in range(4):
        pos = g * 16 + lane
        idxv = jnp.left_shift(jnp.right_shift(pos, 2), 4) + 4 + jnp.bitwise_and(pos, 3)
        condfl[pl.ds(g * 16, 16)] = plsc.load_gather(mstage, [idxv])

    for q in range(ROWS_PER_TILE):
        target = sid + 16 * q
        found = jnp.int32(BIG_I32)
        base = jnp.int32(0)
        for g in range(4):
            fl = condfl[pl.ds(g * 16, 16)]
            fi = (fl > f32(0.5)).astype(jnp.int32)
            cs = lax.cumsum(fi)
            rank = base + cs - fi
            hit = (fi > 0) & (rank == target)
            found = jnp.minimum(found, jnp.min(jnp.where(hit, g * 16 + lane, BIG_I32)))
            base = base + jnp.max(cs)
        rho = found

        @pl.when(rho < BIG_I32)
        def _():
            rglob = cid * 64 + rho
            quarter = VOCAB // 4
            cps = [pltpu.async_copy(
                lp_hbm.at[rglob, pl.ds(p * quarter, quarter)],
                lp_buf.at[pl.ds(p * quarter, quarter)],
                sem if p % 2 == 0 else sem2) for p in range(2)]
            seq_r = seq_v[pl.ds(rglob, 16)][0]

            def blk_max(j, acc16):
                a = [mninf, mninf, mninf, mninf]
                for t in range(C // 64):
                    base2 = j * C + t * 64
                    for u in range(4):
                        a[u] = jnp.maximum(a[u], lp_buf[pl.ds(base2 + u * 16, 16)])
                vm = jnp.maximum(jnp.maximum(a[0], a[1]), jnp.maximum(a[2], a[3]))
                bmj = jnp.max(vm)
                return jnp.where(lane == jnp.bitwise_and(j, 15), bmj, acc16)

            m_nz = f32(NEG_INF)
            for g in range(NB // 16):
                cps[g].wait()
                if g < 2:
                    cps.append(pltpu.async_copy(
                        lp_hbm.at[rglob, pl.ds((g + 2) * quarter, quarter)],
                        lp_buf.at[pl.ds((g + 2) * quarter, quarter)],
                        sem if g % 2 == 0 else sem2))
                if g == 3:
                    tail = lp_buf[pl.ds(VOCAB - 16, 16)]
                    eos_lp = tail[15]
                    lp_buf[pl.ds(VOCAB - 16, 16)] = jnp.where(
                        lane == 15, f32(NEG_INF), tail)
                acc = lax.fori_loop(g * 16, g * 16 + 16, blk_max,
                                    jnp.full((16,), f32(NEG_INF)))
                bm_buf[pl.ds(g * 16, 16)] = seq_r + acc
                m_nz = jnp.maximum(m_nz, jnp.max(acc))

            eosc = jnp.where(eos_lp >= f32(EOS_THRESHOLD) * m_nz,
                             seq_r + eos_lp, f32(MINUS_INF))
            pltpu.sync_copy(bm_buf.at[pl.ds(0, NB)], shared_bm.at[pl.ds(
                pl.multiple_of(rho * NB, NB), NB)])
            misc_buf[...] = jnp.where(lane == 0, eosc, f32(MINUS_INF))
            pltpu.sync_copy(misc_buf, shared_eos.at[pl.ds(
                pl.multiple_of(rho * 16, 16), 16)])

    plsc.subcore_barrier()

    @pl.when(sid < 8)
    def _():
        b = cid * 8 + sid
        pltpu.sync_copy(
            shared_bm.at[pl.ds(pl.multiple_of(sid * (8 * NB), 8 * NB), 8 * NB)],
            bmflat.at[pl.ds(0, 8 * NB)])
        pltpu.sync_copy(shared_misc.at[pl.ds(pl.multiple_of(sid * 32, 32), 32)],
                        condbuf.at[pl.ds(16, 32)])
        pltpu.sync_copy(shared_eos.at[pl.ds(pl.multiple_of(sid * 128, 128), 128)],
                        mstage.at[pl.ds(0, 128)])
        idxe = jnp.minimum(lane, 7) * 16
        eosg = plsc.load_gather(mstage, [idxe])
        eos8 = jnp.where(lane < 8, eosg, f32(NEG_INF))
        bmflat[pl.ds(BEAM * NB, 16)] = eos8
        mrow = jnp.bitwise_and(jnp.right_shift(lane, 2), 1) * 16
        condg = plsc.load_gather(condbuf, [16 + mrow + 4 + jnp.bitwise_and(lane, 3)])
        condbuf[pl.ds(0, 16)] = condg

        def mkid(i, _):
            pos = i * 16 + lane
            jreg = jnp.right_shift(pos, 6)
            kreg = jnp.bitwise_and(pos, 63)
            rid = jreg * VOCAB + kreg * C
            eid = (pos - BEAM * NB) * VOCAB + EOS_INDEX
            idv = jnp.where(pos < BEAM * NB, rid,
                            jnp.where(pos < BEAM * NB + 8, eid, BIG_I32))
            idflat[pl.ds(i * 16, 16)] = idv
            return 0

        lax.fori_loop(0, NC2, mkid, 0)

        selv = jnp.zeros((16,), f32)
        seli = jnp.zeros((16,), jnp.int32)
        for k in range(BEAM):
            def scan2(i, carry):
                bv, bi = carry
                v = bmflat[pl.ds(i * 16, 16)]
                idv = idflat[pl.ds(i * 16, 16)]
                upd = (v > bv) | ((v == bv) & (idv < bi))
                return jnp.where(upd, v, bv), jnp.where(upd, idv, bi)

            bv, bi = lax.fori_loop(0, NC2, scan2,
                                   (jnp.full((16,), f32(NEG_INF)),
                                    jnp.full((16,), BIG_I32)))
            m = jnp.max(bv)
            win = jnp.min(jnp.where(bv == m, bi, BIG_I32))
            selv = jnp.where(lane == k, m, selv)
            seli = jnp.where(lane == k, win, seli)
            off = jnp.bitwise_and(win, VOCAB - 1)
            jw = jnp.right_shift(win, 15)
            pos = jnp.where(off == EOS_INDEX, BEAM * NB + jw,
                            jw * NB + jnp.right_shift(off, 9))
            al = jnp.bitwise_and(pos, ~15)
            q = bmflat[pl.ds(al, 16)]
            bmflat[pl.ds(al, 16)] = jnp.where(lane == jnp.bitwise_and(pos, 15),
                                              f32(NEG_INF), q)
        selv_buf[...] = selv
        seli_buf[...] = seli

        selv_all = selv_buf[...]
        seli_all = seli_buf[...]
        copies = []
        for k in range(BEAM):
            win = seli_all[k]
            jw = jnp.right_shift(win, 15)
            off = jnp.bitwise_and(win, VOCAB - 1)
            start = jnp.left_shift(jnp.right_shift(off, 9), 9)
            copies.append(pltpu.async_copy(
                lp_hbm.at[b * BEAM + jw,
                          pl.ds(pl.multiple_of(start, C), C)],
                gbuf.at[pl.ds(k * C, C)], sem))
        for cp in copies:
            cp.wait()

        sv_reg = mninf
        si_reg = jnp.full((16,), BIG_I32)
        evs = jnp.full((16,), f32(NEG_INF))
        evi = jnp.full((16,), BIG_I32)
        for k in range(BEAM):
            win = seli_all[k]
            wval = selv_all[k]
            jw = jnp.right_shift(win, 15)
            off = jnp.bitwise_and(win, VOCAB - 1)
            iseos = off == EOS_INDEX
            start = jnp.left_shift(jnp.right_shift(off, 9), 9)
            ebf = jnp.where(iseos, f32(1.0), f32(0.0))
            cj = condbuf[pl.ds(jw, 16)][0]
            sj = seq_v[pl.ds(b * BEAM + jw, 16)][0]

            for h in range(2):
                ch = 2 * k + h

                def resc(t, carry):
                    smx, six = carry
                    v = gbuf[pl.ds(k * C + h * CHUNK + t * 16, 16)]
                    posv = start + h * CHUNK + t * 16 + lane
                    sc = cj * (sj + v) - (f32(1.0) - cj) * f32(1e20)
                    sc = jnp.where(posv == EOS_INDEX, f32(NEG_INF), sc)
                    sc = sc - ebf * f32(2e38)
                    idv = jw * VOCAB + posv
                    candv[pl.ds(ch * CHUNK + t * 16, 16)] = sc
                    candi[pl.ds(ch * CHUNK + t * 16, 16)] = idv
                    return _lexmax(smx, six, sc, idv)

                smx, six = lax.fori_loop(0, CHUNK // 16, resc,
                                         (mninf, jnp.full((16,), BIG_I32)))
                chv = jnp.max(smx)
                chi = jnp.min(jnp.where(smx == chv, six, BIG_I32))
                sv_reg = jnp.where(lane == ch, chv, sv_reg)
                si_reg = jnp.where(lane == ch, chi, si_reg)

            evs = jnp.where(lane == k,
                            ebf * wval - (f32(1.0) - ebf) * f32(2e38), evs)
            evi = jnp.where(lane == k, jnp.where(iseos, win, BIG_I32), evi)

        candv[pl.ds(16 * CHUNK, 16)] = evs
        candi[pl.ds(16 * CHUNK, 16)] = evi

        def padc(i, _):
            candv[pl.ds(16 * CHUNK + 16 + i * 16, 16)] = mninf
            candi[pl.ds(16 * CHUNK + 16 + i * 16, 16)] = jnp.full((16,), BIG_I32)
            return 0

        lax.fori_loop(0, (CHUNK - 16) // 16, padc, 0)
        sumv[pl.ds(0, 16)] = sv_reg
        sumi[pl.ds(0, 16)] = si_reg
        ev = jnp.max(evs)
        ei = jnp.min(jnp.where(evs == ev, evi, BIG_I32))
        sumv[pl.ds(16, 16)] = jnp.where(lane == 0, ev, mninf)
        sumi[pl.ds(16, 16)] = jnp.where(lane == 0, ei, jnp.full((16,), BIG_I32))

        ots = jnp.zeros((16,), f32)
        otok = jnp.zeros((16,), jnp.int32)
        opred = jnp.zeros((16,), jnp.int32)
        pv = f32(float("inf"))
        pi = jnp.int32(-1)
        for k in range(BEAM):
            s0 = sumv[pl.ds(0, 16)]
            i0 = sumi[pl.ds(0, 16)]
            s1 = sumv[pl.ds(16, 16)]
            i1 = sumi[pl.ds(16, 16)]
            p0 = lane
            p1 = lane + 16
            bsv, bsi = _lexmax(s0, i0, s1, i1)
            bsp = jnp.where((s1 > s0) | ((s1 == s0) & (i1 < i0)), p1, p0)
            m = jnp.max(bsv)
            wi = jnp.min(jnp.where(bsv == m, bsi, BIG_I32))
            cw = jnp.min(jnp.where((bsv == m) & (bsi == wi), bsp, BIG_I32))
            ots = jnp.where(lane == k, m, ots)
            otok = jnp.where(lane == k, jnp.bitwise_and(wi, VOCAB - 1), otok)
            opred = jnp.where(lane == k, jnp.right_shift(wi, 15) + b * BEAM, opred)
            pv = m
            pi = wi

            cbase = cw * CHUNK

            def rescan(t, carry):
                bv2, bi2 = carry
                idxv = cbase + t * 16 + lane
                v = plsc.load_gather(candv, [idxv])
                idv = plsc.load_gather(candi, [idxv])
                elig = (v < pv) | ((v == pv) & (idv > pi))
                upd = elig & ((v > bv2) | ((v == bv2) & (idv < bi2)))
                return jnp.where(upd, v, bv2), jnp.where(upd, idv, bi2)

            bv2, bi2 = lax.fori_loop(0, CHUNK // 16, rescan,
                                     (mninf, jnp.full((16,), BIG_I32)))
            nv = jnp.max(bv2)
            ni = jnp.min(jnp.where(bv2 == nv, bi2, BIG_I32))
            hit0 = (cw < 16) & (lane == cw)
            hit1 = (cw >= 16) & (lane == cw - 16)
            q0v = sumv[pl.ds(0, 16)]
            q0i = sumi[pl.ds(0, 16)]
            sumv[pl.ds(0, 16)] = jnp.where(hit0, nv, q0v)
            sumi[pl.ds(0, 16)] = jnp.where(hit0, ni, q0i)
            q1v = sumv[pl.ds(16, 16)]
            q1i = sumi[pl.ds(16, 16)]
            sumv[pl.ds(16, 16)] = jnp.where(hit1, nv, q1v)
            sumi[pl.ds(16, 16)] = jnp.where(hit1, ni, q1i)

        ob_ts[...] = ots
        ob_tok[...] = otok
        ob_pred[...] = opred
        ob = pl.ds(pl.multiple_of(b * BEAM, BEAM), BEAM)
        pltpu.sync_copy(ob_ts.at[pl.ds(0, BEAM)], ts_hbm.at[ob])
        pltpu.sync_copy(ob_tok.at[pl.ds(0, BEAM)], tok_hbm.at[ob])
        pltpu.sync_copy(ob_pred.at[pl.ds(0, BEAM)], pred_hbm.at[ob])


@jax.jit
def kernel(log_probs, attn, prev_attn_peak, sequence_scores):
    mesh = plsc.VectorSubcoreMesh(core_axis_name="c", subcore_axis_name="s")
    run = pl.kernel(
        _sc_body,
        out_type=[
            jax.ShapeDtypeStruct((BATCH * BEAM,), jnp.float32),
            jax.ShapeDtypeStruct((BATCH * BEAM,), jnp.int32),
            jax.ShapeDtypeStruct((BATCH * BEAM,), jnp.int32),
        ],
        mesh=mesh,
        compiler_params=pltpu.CompilerParams(needs_layout_passes=False),
        scratch_types=[
            pltpu.VMEM((BATCH * BEAM + 16,), jnp.float32),
            pltpu.VMEM((BATCH * BEAM + 16,), jnp.float32),
            pltpu.VMEM((ROWS_PER_TILE, ENC_LEN), jnp.float32),
            pltpu.VMEM((VOCAB,), jnp.float32),
            pltpu.VMEM((ROWS_PER_TILE * NB,), jnp.float32),
            pltpu.VMEM((16,), jnp.float32),
            pltpu.VMEM_SHARED((64 * NB,), jnp.float32),
            pltpu.VMEM_SHARED((256,), jnp.float32),
            pltpu.VMEM_SHARED((1024,), jnp.float32),
            pltpu.VMEM((256,), jnp.float32),
            pltpu.VMEM((64,), jnp.float32),
            pltpu.VMEM((NCAND,), jnp.float32),
            pltpu.VMEM((NCAND,), jnp.int32),
            pltpu.VMEM((48,), jnp.float32),
            pltpu.VMEM((16,), jnp.float32),
            pltpu.VMEM((16,), jnp.int32),
            pltpu.VMEM((BEAM * C,), jnp.float32),
            pltpu.VMEM((NFIN,), jnp.float32),
            pltpu.VMEM((NFIN,), jnp.int32),
            pltpu.VMEM((48,), jnp.float32),
            pltpu.VMEM((48,), jnp.int32),
            pltpu.VMEM((16,), jnp.float32),
            pltpu.VMEM((16,), jnp.int32),
            pltpu.VMEM((16,), jnp.int32),
            pltpu.SemaphoreType.DMA,
            pltpu.SemaphoreType.DMA,
        ],
    )
    ts, tok, pred = run(log_probs, attn, prev_attn_peak, sequence_scores)
    return (ts.reshape(BATCH, BEAM),
            tok.reshape(BATCH, BEAM),
            pred.reshape(BATCH, BEAM))
